# baseline (speedup 1.0000x reference)
import functools

import jax
import jax.numpy as jnp
from jax import lax
from jax.experimental import pallas as pl
from jax.experimental.pallas import tpu as pltpu
from jax.experimental.pallas import tpu_sc as plsc

NC = 2
NS = 16
NW = NC * NS
CHUNK = 128
GI = 8
CB = 128


def _slab_chunks(rps):
  chunks = []
  off = 0
  while off < rps:
    sz = min(CHUNK, rps - off)
    chunks.append((off, sz))
    off += sz
  return chunks


def _fill(buf, rows_n, cols, value):
  vec = jnp.full((16,), value, jnp.float32)

  @pl.loop(0, rows_n)
  def _(i):
    for j in range(cols // 16):
      buf[i, pl.ds(j * 16, 16)] = vec


def _make_sc_agg(n_pad, d, k_steps):
  rps = n_pad // NS
  chunks = _slab_chunks(rps)
  assert k_steps % GI == 0
  ng = k_steps // GI

  mesh = plsc.VectorSubcoreMesh(core_axis_name="c", subcore_axis_name="s")

  @functools.partial(
      pl.kernel, mesh=mesh,
      out_type=jax.ShapeDtypeStruct((NC, n_pad, d), jnp.float32),
      scratch_types=(
          pltpu.VMEM_SHARED((n_pad, d), jnp.float32),
          pltpu.VMEM((CHUNK,), jnp.int32),
          pltpu.VMEM((CHUNK,), jnp.int32),
          pltpu.VMEM((CHUNK, d), jnp.float32),
          pltpu.SemaphoreType.DMA,
      ),
  )
  def fn(feats, sdi, part, sh_agg, idx_s, idx_d, rows, sem):
    c = lax.axis_index("c")
    s = lax.axis_index("s")
    wid = s * NC + c
    base = s * rps

    _fill(rows, CHUNK, d, 0.0)
    for off, sz in chunks:
      pltpu.sync_copy(rows.at[pl.ds(0, sz)], sh_agg.at[pl.ds(base + off, sz)])
    plsc.subcore_barrier()

    @pl.loop(0, k_steps)
    def _(k):
      pltpu.sync_copy(sdi.at[wid, 2 * k], idx_s)
      pltpu.sync_copy(sdi.at[wid, 2 * k + 1], idx_d)
      pltpu.async_copy(feats.at[idx_s], rows, sem).wait()
      pltpu.sync_copy(rows, sh_agg.at[idx_d], add=True)

    plsc.subcore_barrier()

    for off, sz in chunks:
      pltpu.sync_copy(sh_agg.at[pl.ds(base + off, sz)], rows.at[pl.ds(0, sz)])
      pltpu.sync_copy(rows.at[pl.ds(0, sz)], part.at[c, pl.ds(base + off, sz)])

  return fn


def _make_sc_cnt(n_pad, k_steps):
  rps = n_pad // NS
  chunks = _slab_chunks(rps)
  assert k_steps % GI == 0
  ng = k_steps // GI

  mesh = plsc.VectorSubcoreMesh(core_axis_name="c", subcore_axis_name="s")

  @functools.partial(
      pl.kernel, mesh=mesh,
      out_type=jax.ShapeDtypeStruct((NC, n_pad, CB), jnp.float32),
      scratch_types=(
          pltpu.VMEM_SHARED((n_pad, CB), jnp.float32),
          pltpu.VMEM((CHUNK,), jnp.int32),
          pltpu.VMEM((CHUNK, CB), jnp.float32),
      ),
  )
  def fn(sdi, cntp, sh_cnt, idx_d, ones_v):
    c = lax.axis_index("c")
    s = lax.axis_index("s")
    wid = s * NC + c
    base = s * rps

    _fill(ones_v, CHUNK, CB, 0.0)
    for off, sz in chunks:
      pltpu.sync_copy(ones_v.at[pl.ds(0, sz)], sh_cnt.at[pl.ds(base + off, sz)])
    _fill(ones_v, CHUNK, CB, 1.0)
    plsc.subcore_barrier()

    @pl.loop(0, k_steps)
    def _(k):
      pltpu.sync_copy(sdi.at[wid, 2 * k + 1], idx_d)
      pltpu.sync_copy(ones_v, sh_cnt.at[idx_d], add=True)

    plsc.subcore_barrier()
    for off, sz in chunks:
      pltpu.sync_copy(sh_cnt.at[pl.ds(base + off, sz)], ones_v.at[pl.ds(0, sz)])
      pltpu.sync_copy(ones_v.at[pl.ds(0, sz)], cntp.at[c, pl.ds(base + off, sz)])

  return fn


def _tc_dense(part, cnt3, x, w_l, w_r, b, *, relu, n, d):
  rblk = 1000
  grid = (n // rblk,)

  def body(part_ref, cnt_ref, x_ref, wl_ref, wr_ref, b_ref, o_ref):
    p = part_ref[0] + part_ref[1]
    cn = cnt_ref[0, :, 0:1] + cnt_ref[1, :, 0:1]
    mean = p / jnp.maximum(cn, 1.0)
    acc = (jnp.dot(mean, wl_ref[...], preferred_element_type=jnp.float32)
           + jnp.dot(x_ref[...], wr_ref[...], preferred_element_type=jnp.float32)
           + b_ref[...])
    if relu:
      acc = jnp.maximum(acc, 0.0)
    o_ref[...] = acc

  return pl.pallas_call(
      body,
      grid=grid,
      in_specs=[
          pl.BlockSpec((NC, rblk, d), lambda i: (0, i, 0)),
          pl.BlockSpec((NC, rblk, CB), lambda i: (0, i, 0)),
          pl.BlockSpec((rblk, d), lambda i: (i, 0)),
          pl.BlockSpec((d, d), lambda i: (0, 0)),
          pl.BlockSpec((d, d), lambda i: (0, 0)),
          pl.BlockSpec((1, d), lambda i: (0, 0)),
      ],
      out_specs=pl.BlockSpec((rblk, d), lambda i: (i, 0)),
      out_shape=jax.ShapeDtypeStruct((n, d), jnp.float32),
  )(part, cnt3, x, w_l, w_r, b)


def kernel(x, edge_index, node_features, W1_l, b1, W1_r, W2_l, b2, W2_r):
  del x
  n, d = node_features.shape
  e = edge_index.shape[1]

  n_pad = -(-(n + 1) // (NS * 32)) * (NS * 32)
  k_steps = -(-e // (NW * CHUNK))
  k_steps = -(-k_steps // GI) * GI
  e_pad = NW * k_steps * CHUNK

  src = edge_index[0]
  dst = edge_index[1]
  pad = e_pad - e
  srcp = jnp.concatenate([src, jnp.zeros((pad,), jnp.int32)]).reshape(
      NW, k_steps, CHUNK)
  dstp = jnp.concatenate([dst, jnp.full((pad,), n, jnp.int32)]).reshape(
      NW, k_steps, CHUNK)
  sdi = jnp.stack([srcp, dstp], axis=2).reshape(NW, 2 * k_steps, CHUNK)

  agg = _make_sc_agg(n_pad, d, k_steps)
  cntp = _make_sc_cnt(n_pad, k_steps)(sdi)

  part1 = agg(node_features, sdi)
  b1r = b1.reshape(1, d)
  b2r = b2.reshape(1, d)
  h1 = _tc_dense(part1, cntp, node_features, W1_l, W1_r, b1r,
                 relu=True, n=n, d=d)
  part2 = agg(h1, sdi)
  out = _tc_dense(part2, cntp, h1, W2_l, W2_r, b2r, relu=False, n=n, d=d)
  return out


# speedup vs baseline: 1.3982x; 1.3982x over previous
import functools

import jax
import jax.numpy as jnp
from jax import lax
from jax.experimental import pallas as pl
from jax.experimental.pallas import tpu as pltpu
from jax.experimental.pallas import tpu_sc as plsc

NC = 2
NS = 16
NW = NC * NS
CHUNK = 128
GI = 8
CB = 128


def _slab_chunks(rps):
  chunks = []
  off = 0
  while off < rps:
    sz = min(CHUNK, rps - off)
    chunks.append((off, sz))
    off += sz
  return chunks


def _fill(buf, rows_n, cols, value):
  vec = jnp.full((16,), value, jnp.float32)

  @pl.loop(0, rows_n)
  def _(i):
    for j in range(cols // 16):
      buf[i, pl.ds(j * 16, 16)] = vec


def _make_sc_agg(n_pad, d, k_steps):
  rps = n_pad // NS
  chunks = _slab_chunks(rps)

  mesh = plsc.VectorSubcoreMesh(core_axis_name="c", subcore_axis_name="s")

  @functools.partial(
      pl.kernel, mesh=mesh,
      out_type=jax.ShapeDtypeStruct((NC, n_pad, d), jnp.float32),
      scratch_types=(
          pltpu.VMEM_SHARED((n_pad, d), jnp.float32),
          pltpu.VMEM((CHUNK,), jnp.int32),
          pltpu.VMEM((CHUNK,), jnp.int32),
          pltpu.VMEM((CHUNK, d), jnp.float32),
          pltpu.SemaphoreType.DMA,
      ),
  )
  def fn(feats, sdi, part, sh_agg, idx_s, idx_d, rows, sem):
    c = lax.axis_index("c")
    s = lax.axis_index("s")
    wid = s * NC + c
    base = s * rps

    _fill(rows, CHUNK, d, 0.0)
    for off, sz in chunks:
      pltpu.sync_copy(rows.at[pl.ds(0, sz)], sh_agg.at[pl.ds(base + off, sz)])
    plsc.subcore_barrier()

    @pl.loop(0, k_steps)
    def _(k):
      pltpu.sync_copy(sdi.at[wid, 2 * k], idx_s)
      pltpu.sync_copy(sdi.at[wid, 2 * k + 1], idx_d)
      pltpu.async_copy(feats.at[idx_s], rows, sem).wait()
      pltpu.sync_copy(rows, sh_agg.at[idx_d], add=True)

    plsc.subcore_barrier()

    for off, sz in chunks:
      pltpu.sync_copy(sh_agg.at[pl.ds(base + off, sz)], rows.at[pl.ds(0, sz)])
      pltpu.sync_copy(rows.at[pl.ds(0, sz)], part.at[c, pl.ds(base + off, sz)])

  return fn


def _make_sc_cnt(n_pad, k_steps):
  rps = n_pad // NS
  chunks = _slab_chunks(rps)

  mesh = plsc.VectorSubcoreMesh(core_axis_name="c", subcore_axis_name="s")

  @functools.partial(
      pl.kernel, mesh=mesh,
      out_type=jax.ShapeDtypeStruct((NC, n_pad, CB), jnp.float32),
      scratch_types=(
          pltpu.VMEM_SHARED((n_pad, CB), jnp.float32),
          pltpu.VMEM((CHUNK,), jnp.int32),
          pltpu.VMEM((CHUNK, CB), jnp.float32),
      ),
  )
  def fn(sdi, cntp, sh_cnt, idx_d, ones_v):
    c = lax.axis_index("c")
    s = lax.axis_index("s")
    wid = s * NC + c
    base = s * rps

    _fill(ones_v, CHUNK, CB, 0.0)
    for off, sz in chunks:
      pltpu.sync_copy(ones_v.at[pl.ds(0, sz)], sh_cnt.at[pl.ds(base + off, sz)])
    _fill(ones_v, CHUNK, CB, 1.0)
    plsc.subcore_barrier()

    @pl.loop(0, k_steps)
    def _(k):
      pltpu.sync_copy(sdi.at[wid, 2 * k + 1], idx_d)
      pltpu.sync_copy(ones_v, sh_cnt.at[idx_d], add=True)

    plsc.subcore_barrier()
    for off, sz in chunks:
      pltpu.sync_copy(sh_cnt.at[pl.ds(base + off, sz)], ones_v.at[pl.ds(0, sz)])
      pltpu.sync_copy(ones_v.at[pl.ds(0, sz)], cntp.at[c, pl.ds(base + off, sz)])

  return fn


def _tc_dense(part, cnt3, x, w_l, w_r, b, *, relu, n, d):
  rblk = 1000
  grid = (n // rblk,)

  def body(part_ref, cnt_ref, x_ref, wl_ref, wr_ref, b_ref, o_ref):
    p = part_ref[0] + part_ref[1]
    cn = cnt_ref[0, :, 0:1] + cnt_ref[1, :, 0:1]
    mean = p / jnp.maximum(cn, 1.0)
    acc = (jnp.dot(mean, wl_ref[...], preferred_element_type=jnp.float32)
           + jnp.dot(x_ref[...], wr_ref[...], preferred_element_type=jnp.float32)
           + b_ref[...])
    if relu:
      acc = jnp.maximum(acc, 0.0)
    o_ref[...] = acc

  return pl.pallas_call(
      body,
      grid=grid,
      in_specs=[
          pl.BlockSpec((NC, rblk, d), lambda i: (0, i, 0)),
          pl.BlockSpec((NC, rblk, CB), lambda i: (0, i, 0)),
          pl.BlockSpec((rblk, d), lambda i: (i, 0)),
          pl.BlockSpec((d, d), lambda i: (0, 0)),
          pl.BlockSpec((d, d), lambda i: (0, 0)),
          pl.BlockSpec((1, d), lambda i: (0, 0)),
      ],
      out_specs=pl.BlockSpec((rblk, d), lambda i: (i, 0)),
      out_shape=jax.ShapeDtypeStruct((n, d), jnp.float32),
  )(part, cnt3, x, w_l, w_r, b)


def kernel(x, edge_index, node_features, W1_l, b1, W1_r, W2_l, b2, W2_r):
  del x
  n, d = node_features.shape
  e = edge_index.shape[1]

  n_pad = -(-(n + 1) // (NS * 32)) * (NS * 32)
  k_steps = -(-e // (NW * CHUNK))
  e_pad = NW * k_steps * CHUNK

  src = edge_index[0]
  dst = edge_index[1]
  pad = e_pad - e
  trash = n + (jnp.arange(pad, dtype=jnp.int32) % 128)
  srcp = jnp.concatenate([src, jnp.zeros((pad,), jnp.int32)]).reshape(
      k_steps, NW, CHUNK).transpose(1, 0, 2)
  dstp = jnp.concatenate([dst, trash]).reshape(
      k_steps, NW, CHUNK).transpose(1, 0, 2)
  sdi = jnp.stack([srcp, dstp], axis=2).reshape(NW, 2 * k_steps, CHUNK)

  agg = _make_sc_agg(n_pad, d, k_steps)
  cntp = _make_sc_cnt(n_pad, k_steps)(sdi)

  part1 = agg(node_features, sdi)
  b1r = b1.reshape(1, d)
  b2r = b2.reshape(1, d)
  h1 = _tc_dense(part1, cntp, node_features, W1_l, W1_r, b1r,
                 relu=True, n=n, d=d)
  part2 = agg(h1, sdi)
  out = _tc_dense(part2, cntp, h1, W2_l, W2_r, b2r, relu=False, n=n, d=d)
  return out


# speedup vs baseline: 1.3993x; 1.0008x over previous
import functools

import jax
import jax.numpy as jnp
from jax import lax
from jax.experimental import pallas as pl
from jax.experimental.pallas import tpu as pltpu
from jax.experimental.pallas import tpu_sc as plsc

NC = 2
NS = 16
NW = NC * NS
CHUNK = 128
GI = 8
CB = 128


def _slab_chunks(rps):
  chunks = []
  off = 0
  while off < rps:
    sz = min(CHUNK, rps - off)
    chunks.append((off, sz))
    off += sz
  return chunks


def _fill(buf, rows_n, cols, value):
  vec = jnp.full((16,), value, jnp.float32)

  @pl.loop(0, rows_n)
  def _(i):
    for j in range(cols // 16):
      buf[i, pl.ds(j * 16, 16)] = vec


def _make_sc_agg(n_pad, d, k_steps):
  rps = n_pad // NS
  chunks = _slab_chunks(rps)

  mesh = plsc.VectorSubcoreMesh(core_axis_name="c", subcore_axis_name="s")

  @functools.partial(
      pl.kernel, mesh=mesh,
      out_type=jax.ShapeDtypeStruct((NC, n_pad, d), jnp.float32),
      scratch_types=(
          pltpu.VMEM_SHARED((n_pad, d), jnp.float32),
          pltpu.VMEM((CHUNK,), jnp.int32),
          pltpu.VMEM((CHUNK,), jnp.int32),
          pltpu.VMEM((CHUNK, d), jnp.float32),
          pltpu.SemaphoreType.DMA,
      ),
  )
  def fn(feats, srci, dsti, part, sh_agg, idx_s, idx_d, rows, sem):
    c = lax.axis_index("c")
    s = lax.axis_index("s")
    wid = s * NC + c
    base = s * rps

    _fill(rows, CHUNK, d, 0.0)
    for off, sz in chunks:
      pltpu.sync_copy(rows.at[pl.ds(0, sz)], sh_agg.at[pl.ds(base + off, sz)])
    plsc.subcore_barrier()

    @pl.loop(0, k_steps)
    def _(k):
      pltpu.sync_copy(srci.at[wid, k], idx_s)
      pltpu.sync_copy(dsti.at[wid, k], idx_d)
      pltpu.async_copy(feats.at[idx_s], rows, sem).wait()
      pltpu.sync_copy(rows, sh_agg.at[idx_d], add=True)

    plsc.subcore_barrier()

    for off, sz in chunks:
      pltpu.sync_copy(sh_agg.at[pl.ds(base + off, sz)], rows.at[pl.ds(0, sz)])
      pltpu.sync_copy(rows.at[pl.ds(0, sz)], part.at[c, pl.ds(base + off, sz)])

  return fn


def _make_sc_cnt(n_pad, k_steps):
  rps = n_pad // NS
  chunks = _slab_chunks(rps)

  mesh = plsc.VectorSubcoreMesh(core_axis_name="c", subcore_axis_name="s")

  @functools.partial(
      pl.kernel, mesh=mesh,
      out_type=jax.ShapeDtypeStruct((NC, n_pad, CB), jnp.float32),
      scratch_types=(
          pltpu.VMEM_SHARED((n_pad, CB), jnp.float32),
          pltpu.VMEM((CHUNK,), jnp.int32),
          pltpu.VMEM((CHUNK, CB), jnp.float32),
      ),
  )
  def fn(dsti, cntp, sh_cnt, idx_d, ones_v):
    c = lax.axis_index("c")
    s = lax.axis_index("s")
    wid = s * NC + c
    base = s * rps

    _fill(ones_v, CHUNK, CB, 0.0)
    for off, sz in chunks:
      pltpu.sync_copy(ones_v.at[pl.ds(0, sz)], sh_cnt.at[pl.ds(base + off, sz)])
    _fill(ones_v, CHUNK, CB, 1.0)
    plsc.subcore_barrier()

    @pl.loop(0, k_steps)
    def _(k):
      pltpu.sync_copy(dsti.at[wid, k], idx_d)
      pltpu.sync_copy(ones_v, sh_cnt.at[idx_d], add=True)

    plsc.subcore_barrier()
    for off, sz in chunks:
      pltpu.sync_copy(sh_cnt.at[pl.ds(base + off, sz)], ones_v.at[pl.ds(0, sz)])
      pltpu.sync_copy(ones_v.at[pl.ds(0, sz)], cntp.at[c, pl.ds(base + off, sz)])

  return fn


def _tc_dense(part, cnt3, x, w_l, w_r, b, *, relu, n, d):
  rblk = 1000
  grid = (n // rblk,)

  def body(part_ref, cnt_ref, x_ref, wl_ref, wr_ref, b_ref, o_ref):
    p = part_ref[0] + part_ref[1]
    cn = cnt_ref[0, :, 0:1] + cnt_ref[1, :, 0:1]
    mean = p / jnp.maximum(cn, 1.0)
    acc = (jnp.dot(mean, wl_ref[...], preferred_element_type=jnp.float32)
           + jnp.dot(x_ref[...], wr_ref[...], preferred_element_type=jnp.float32)
           + b_ref[...])
    if relu:
      acc = jnp.maximum(acc, 0.0)
    o_ref[...] = acc

  return pl.pallas_call(
      body,
      grid=grid,
      in_specs=[
          pl.BlockSpec((NC, rblk, d), lambda i: (0, i, 0)),
          pl.BlockSpec((NC, rblk, CB), lambda i: (0, i, 0)),
          pl.BlockSpec((rblk, d), lambda i: (i, 0)),
          pl.BlockSpec((d, d), lambda i: (0, 0)),
          pl.BlockSpec((d, d), lambda i: (0, 0)),
          pl.BlockSpec((1, d), lambda i: (0, 0)),
      ],
      out_specs=pl.BlockSpec((rblk, d), lambda i: (i, 0)),
      out_shape=jax.ShapeDtypeStruct((n, d), jnp.float32),
  )(part, cnt3, x, w_l, w_r, b)


def kernel(x, edge_index, node_features, W1_l, b1, W1_r, W2_l, b2, W2_r):
  del x
  n, d = node_features.shape
  e = edge_index.shape[1]

  n_pad = -(-(n + 1) // (NS * 32)) * (NS * 32)
  k_steps = -(-e // (NW * CHUNK))
  e_pad = NW * k_steps * CHUNK

  src = edge_index[0]
  dst = edge_index[1]
  pad = e_pad - e
  if e % NW == 0:
    epw = e // NW
    ppw = pad // NW
    trash = (n + (jnp.arange(NW * ppw, dtype=jnp.int32) % 128)).reshape(NW, ppw)
    srcp = jnp.concatenate(
        [src.reshape(NW, epw), jnp.zeros((NW, ppw), jnp.int32)],
        axis=1).reshape(NW, k_steps, CHUNK)
    dstp = jnp.concatenate(
        [dst.reshape(NW, epw), trash], axis=1).reshape(NW, k_steps, CHUNK)
  else:
    trash = n + (jnp.arange(pad, dtype=jnp.int32) % 128)
    srcp = jnp.concatenate([src, jnp.zeros((pad,), jnp.int32)]).reshape(
        NW, k_steps, CHUNK)
    dstp = jnp.concatenate([dst, trash]).reshape(NW, k_steps, CHUNK)
  agg = _make_sc_agg(n_pad, d, k_steps)
  cntp = _make_sc_cnt(n_pad, k_steps)(dstp)

  part1 = agg(node_features, srcp, dstp)
  b1r = b1.reshape(1, d)
  b2r = b2.reshape(1, d)
  h1 = _tc_dense(part1, cntp, node_features, W1_l, W1_r, b1r,
                 relu=True, n=n, d=d)
  part2 = agg(h1, srcp, dstp)
  out = _tc_dense(part2, cntp, h1, W2_l, W2_r, b2r, relu=False, n=n, d=d)
  return out


# speedup vs baseline: 1.7682x; 1.2637x over previous
import functools

import jax
import jax.numpy as jnp
from jax import lax
from jax.experimental import pallas as pl
from jax.experimental.pallas import tpu as pltpu
from jax.experimental.pallas import tpu_sc as plsc

NC = 2
NS = 16
NW = NC * NS
CHUNK = 128
GI = 8
CB = 128


def _slab_chunks(rps):
  chunks = []
  off = 0
  while off < rps:
    sz = min(CHUNK, rps - off)
    chunks.append((off, sz))
    off += sz
  return chunks


def _fill(buf, rows_n, cols, value):
  vec = jnp.full((16,), value, jnp.float32)

  @pl.loop(0, rows_n)
  def _(i):
    for j in range(cols // 16):
      buf[i, pl.ds(j * 16, 16)] = vec


def _make_sc_agg(n_pad, d, k_steps):
  rps = n_pad // NS
  chunks = _slab_chunks(rps)

  mesh = plsc.VectorSubcoreMesh(core_axis_name="c", subcore_axis_name="s")

  @functools.partial(
      pl.kernel, mesh=mesh,
      out_type=jax.ShapeDtypeStruct((NC, n_pad, d), jnp.float32),
      scratch_types=(
          pltpu.VMEM_SHARED((n_pad, d), jnp.float32),
          pltpu.VMEM((2, CHUNK), jnp.int32),
          pltpu.VMEM((CHUNK,), jnp.int32),
          pltpu.VMEM((2, CHUNK, d), jnp.float32),
          pltpu.SemaphoreType.DMA,
      ),
  )
  def fn(feats, srci, dsti, part, sh_agg, idx_s, idx_d, rows2, sem):
    c = lax.axis_index("c")
    s = lax.axis_index("s")
    wid = s * NC + c
    base = s * rps

    _fill(rows2.at[0], CHUNK, d, 0.0)
    for off, sz in chunks:
      pltpu.sync_copy(rows2.at[0, pl.ds(0, sz)],
                      sh_agg.at[pl.ds(base + off, sz)])
    plsc.subcore_barrier()

    pltpu.sync_copy(srci.at[wid, 0], idx_s.at[0])
    pltpu.async_copy(feats.at[idx_s.at[0]], rows2.at[0], sem)

    @pl.loop(0, k_steps)
    def _(k):
      par = lax.rem(k, 2)
      nxt = lax.rem(k + 1, 2)

      @pl.when(k + 1 < k_steps)
      def _():
        pltpu.sync_copy(srci.at[wid, k + 1], idx_s.at[nxt])
        pltpu.async_copy(feats.at[idx_s.at[nxt]], rows2.at[nxt], sem)

      pltpu.sync_copy(dsti.at[wid, k], idx_d)
      pltpu.make_async_copy(feats.at[idx_s.at[0]], rows2.at[par], sem).wait()
      pltpu.sync_copy(rows2.at[par], sh_agg.at[idx_d], add=True)

    plsc.subcore_barrier()

    for off, sz in chunks:
      pltpu.sync_copy(sh_agg.at[pl.ds(base + off, sz)],
                      rows2.at[0, pl.ds(0, sz)])
      pltpu.sync_copy(rows2.at[0, pl.ds(0, sz)],
                      part.at[c, pl.ds(base + off, sz)])

  return fn


def _make_sc_cnt(n_pad, k_steps):
  rps = n_pad // NS
  chunks = _slab_chunks(rps)

  mesh = plsc.VectorSubcoreMesh(core_axis_name="c", subcore_axis_name="s")

  @functools.partial(
      pl.kernel, mesh=mesh,
      out_type=jax.ShapeDtypeStruct((NC, n_pad, CB), jnp.float32),
      scratch_types=(
          pltpu.VMEM_SHARED((n_pad, CB), jnp.float32),
          pltpu.VMEM((CHUNK,), jnp.int32),
          pltpu.VMEM((CHUNK, CB), jnp.float32),
      ),
  )
  def fn(dsti, cntp, sh_cnt, idx_d, ones_v):
    c = lax.axis_index("c")
    s = lax.axis_index("s")
    wid = s * NC + c
    base = s * rps

    _fill(ones_v, CHUNK, CB, 0.0)
    for off, sz in chunks:
      pltpu.sync_copy(ones_v.at[pl.ds(0, sz)], sh_cnt.at[pl.ds(base + off, sz)])
    _fill(ones_v, CHUNK, CB, 1.0)
    plsc.subcore_barrier()

    @pl.loop(0, k_steps)
    def _(k):
      pltpu.sync_copy(dsti.at[wid, k], idx_d)
      pltpu.sync_copy(ones_v, sh_cnt.at[idx_d], add=True)

    plsc.subcore_barrier()
    for off, sz in chunks:
      pltpu.sync_copy(sh_cnt.at[pl.ds(base + off, sz)], ones_v.at[pl.ds(0, sz)])
      pltpu.sync_copy(ones_v.at[pl.ds(0, sz)], cntp.at[c, pl.ds(base + off, sz)])

  return fn


def _tc_dense(part, cnt3, x, w_l, w_r, b, *, relu, n, d):
  rblk = 1000
  grid = (n // rblk,)

  def body(part_ref, cnt_ref, x_ref, wl_ref, wr_ref, b_ref, o_ref):
    p = part_ref[0] + part_ref[1]
    cn = cnt_ref[0, :, 0:1] + cnt_ref[1, :, 0:1]
    mean = p / jnp.maximum(cn, 1.0)
    acc = (jnp.dot(mean, wl_ref[...], preferred_element_type=jnp.float32)
           + jnp.dot(x_ref[...], wr_ref[...], preferred_element_type=jnp.float32)
           + b_ref[...])
    if relu:
      acc = jnp.maximum(acc, 0.0)
    o_ref[...] = acc

  return pl.pallas_call(
      body,
      grid=grid,
      in_specs=[
          pl.BlockSpec((NC, rblk, d), lambda i: (0, i, 0)),
          pl.BlockSpec((NC, rblk, CB), lambda i: (0, i, 0)),
          pl.BlockSpec((rblk, d), lambda i: (i, 0)),
          pl.BlockSpec((d, d), lambda i: (0, 0)),
          pl.BlockSpec((d, d), lambda i: (0, 0)),
          pl.BlockSpec((1, d), lambda i: (0, 0)),
      ],
      out_specs=pl.BlockSpec((rblk, d), lambda i: (i, 0)),
      out_shape=jax.ShapeDtypeStruct((n, d), jnp.float32),
  )(part, cnt3, x, w_l, w_r, b)


def kernel(x, edge_index, node_features, W1_l, b1, W1_r, W2_l, b2, W2_r):
  del x
  n, d = node_features.shape
  e = edge_index.shape[1]

  n_pad = -(-(n + 1) // (NS * 32)) * (NS * 32)
  k_steps = -(-e // (NW * CHUNK))
  e_pad = NW * k_steps * CHUNK

  src = edge_index[0]
  dst = edge_index[1]
  pad = e_pad - e
  if e % NW == 0:
    epw = e // NW
    ppw = pad // NW
    trash = (n + (jnp.arange(NW * ppw, dtype=jnp.int32) % 128)).reshape(NW, ppw)
    srcp = jnp.concatenate(
        [src.reshape(NW, epw), jnp.zeros((NW, ppw), jnp.int32)],
        axis=1).reshape(NW, k_steps, CHUNK)
    dstp = jnp.concatenate(
        [dst.reshape(NW, epw), trash], axis=1).reshape(NW, k_steps, CHUNK)
  else:
    trash = n + (jnp.arange(pad, dtype=jnp.int32) % 128)
    srcp = jnp.concatenate([src, jnp.zeros((pad,), jnp.int32)]).reshape(
        NW, k_steps, CHUNK)
    dstp = jnp.concatenate([dst, trash]).reshape(NW, k_steps, CHUNK)
  agg = _make_sc_agg(n_pad, d, k_steps)
  cntp = _make_sc_cnt(n_pad, k_steps)(dstp)

  part1 = agg(node_features, srcp, dstp)
  b1r = b1.reshape(1, d)
  b2r = b2.reshape(1, d)
  h1 = _tc_dense(part1, cntp, node_features, W1_l, W1_r, b1r,
                 relu=True, n=n, d=d)
  part2 = agg(h1, srcp, dstp)
  out = _tc_dense(part2, cntp, h1, W2_l, W2_r, b2r, relu=False, n=n, d=d)
  return out


# speedup vs baseline: 1.8546x; 1.0488x over previous
import functools

import jax
import jax.numpy as jnp
from jax import lax
from jax.experimental import pallas as pl
from jax.experimental.pallas import tpu as pltpu
from jax.experimental.pallas import tpu_sc as plsc

NC = 2
NS = 16
NW = NC * NS
CHUNK = 128
GI = 8
CB = 128


def _slab_chunks(rps):
  chunks = []
  off = 0
  while off < rps:
    sz = min(CHUNK, rps - off)
    chunks.append((off, sz))
    off += sz
  return chunks


def _fill(buf, rows_n, cols, value):
  vec = jnp.full((16,), value, jnp.float32)

  @pl.loop(0, rows_n)
  def _(i):
    for j in range(cols // 16):
      buf[i, pl.ds(j * 16, 16)] = vec


def _make_sc_agg(n_pad, d, k_steps):
  rps = n_pad // NS
  chunks = _slab_chunks(rps)

  mesh = plsc.VectorSubcoreMesh(core_axis_name="c", subcore_axis_name="s")

  @functools.partial(
      pl.kernel, mesh=mesh,
      out_type=jax.ShapeDtypeStruct((NC, n_pad, d), jnp.float32),
      scratch_types=(
          pltpu.VMEM_SHARED((n_pad, d), jnp.float32),
          pltpu.VMEM((2, CHUNK), jnp.int32),
          pltpu.VMEM((CHUNK,), jnp.int32),
          pltpu.VMEM((2, CHUNK, d), jnp.float32),
          pltpu.SemaphoreType.DMA,
      ),
  )
  def fn(feats, srci, dsti, part, sh_agg, idx_s, idx_d, rows2, sem):
    c = lax.axis_index("c")
    s = lax.axis_index("s")
    wid = s * NC + c
    base = s * rps

    _fill(rows2.at[0], CHUNK, d, 0.0)
    for off, sz in chunks:
      pltpu.sync_copy(rows2.at[0, pl.ds(0, sz)],
                      sh_agg.at[pl.ds(base + off, sz)])
    plsc.subcore_barrier()

    pltpu.sync_copy(srci.at[wid, 0], idx_s.at[0])
    pltpu.async_copy(feats.at[idx_s.at[0]], rows2.at[0], sem)

    @pl.loop(0, k_steps)
    def _(k):
      par = lax.rem(k, 2)
      nxt = lax.rem(k + 1, 2)

      @pl.when(k + 1 < k_steps)
      def _():
        pltpu.sync_copy(srci.at[wid, k + 1], idx_s.at[nxt])
        pltpu.async_copy(feats.at[idx_s.at[nxt]], rows2.at[nxt], sem)

      pltpu.sync_copy(dsti.at[wid, k], idx_d)
      pltpu.make_async_copy(feats.at[idx_s.at[0]], rows2.at[par], sem).wait()
      pltpu.sync_copy(rows2.at[par], sh_agg.at[idx_d], add=True)

    plsc.subcore_barrier()

    for off, sz in chunks:
      pltpu.sync_copy(sh_agg.at[pl.ds(base + off, sz)],
                      rows2.at[0, pl.ds(0, sz)])
      pltpu.sync_copy(rows2.at[0, pl.ds(0, sz)],
                      part.at[c, pl.ds(base + off, sz)])

  return fn


def _make_sc_cnt(n_pad, k_steps):
  rps = n_pad // NS
  chunks = _slab_chunks(rps)

  mesh = plsc.VectorSubcoreMesh(core_axis_name="c", subcore_axis_name="s")

  @functools.partial(
      pl.kernel, mesh=mesh,
      out_type=jax.ShapeDtypeStruct((NC, n_pad, CB), jnp.float32),
      scratch_types=(
          pltpu.VMEM_SHARED((n_pad, CB), jnp.float32),
          pltpu.VMEM((2, CHUNK), jnp.int32),
          pltpu.VMEM((CHUNK, CB), jnp.float32),
          pltpu.SemaphoreType.DMA,
      ),
  )
  def fn(dsti, cntp, sh_cnt, idx_d2, ones_v, sem):
    c = lax.axis_index("c")
    s = lax.axis_index("s")
    wid = s * NC + c
    base = s * rps

    _fill(ones_v, CHUNK, CB, 0.0)
    for off, sz in chunks:
      pltpu.sync_copy(ones_v.at[pl.ds(0, sz)], sh_cnt.at[pl.ds(base + off, sz)])
    _fill(ones_v, CHUNK, CB, 1.0)
    plsc.subcore_barrier()

    pltpu.sync_copy(dsti.at[wid, 0], idx_d2.at[0])

    @pl.loop(0, k_steps)
    def _(k):
      par = lax.rem(k, 2)
      nxt = lax.rem(k + 1, 2)

      @pl.when(k + 1 < k_steps)
      def _():
        pltpu.async_copy(dsti.at[wid, k + 1], idx_d2.at[nxt], sem)

      pltpu.sync_copy(ones_v, sh_cnt.at[idx_d2.at[par]], add=True)

      @pl.when(k + 1 < k_steps)
      def _():
        pltpu.make_async_copy(dsti.at[wid, 0], idx_d2.at[nxt], sem).wait()

    plsc.subcore_barrier()
    for off, sz in chunks:
      pltpu.sync_copy(sh_cnt.at[pl.ds(base + off, sz)], ones_v.at[pl.ds(0, sz)])
      pltpu.sync_copy(ones_v.at[pl.ds(0, sz)], cntp.at[c, pl.ds(base + off, sz)])

  return fn


def _tc_dense(part, cnt3, x, w_l, w_r, b, *, relu, n, d):
  rblk = 1000
  grid = (n // rblk,)

  def body(part_ref, cnt_ref, x_ref, wl_ref, wr_ref, b_ref, o_ref):
    p = part_ref[0] + part_ref[1]
    cn = cnt_ref[0, :, 0:1] + cnt_ref[1, :, 0:1]
    mean = p / jnp.maximum(cn, 1.0)
    acc = (jnp.dot(mean, wl_ref[...], preferred_element_type=jnp.float32)
           + jnp.dot(x_ref[...], wr_ref[...], preferred_element_type=jnp.float32)
           + b_ref[...])
    if relu:
      acc = jnp.maximum(acc, 0.0)
    o_ref[...] = acc

  return pl.pallas_call(
      body,
      grid=grid,
      in_specs=[
          pl.BlockSpec((NC, rblk, d), lambda i: (0, i, 0)),
          pl.BlockSpec((NC, rblk, CB), lambda i: (0, i, 0)),
          pl.BlockSpec((rblk, d), lambda i: (i, 0)),
          pl.BlockSpec((d, d), lambda i: (0, 0)),
          pl.BlockSpec((d, d), lambda i: (0, 0)),
          pl.BlockSpec((1, d), lambda i: (0, 0)),
      ],
      out_specs=pl.BlockSpec((rblk, d), lambda i: (i, 0)),
      out_shape=jax.ShapeDtypeStruct((n, d), jnp.float32),
  )(part, cnt3, x, w_l, w_r, b)


def kernel(x, edge_index, node_features, W1_l, b1, W1_r, W2_l, b2, W2_r):
  del x
  n, d = node_features.shape
  e = edge_index.shape[1]

  n_pad = -(-(n + 1) // (NS * 32)) * (NS * 32)
  k_steps = -(-e // (NW * CHUNK))
  e_pad = NW * k_steps * CHUNK

  src = edge_index[0]
  dst = edge_index[1]
  pad = e_pad - e
  if e % NW == 0:
    epw = e // NW
    ppw = pad // NW
    trash = (n + (jnp.arange(NW * ppw, dtype=jnp.int32) % 128)).reshape(NW, ppw)
    srcp = jnp.concatenate(
        [src.reshape(NW, epw), jnp.zeros((NW, ppw), jnp.int32)],
        axis=1).reshape(NW, k_steps, CHUNK)
    dstp = jnp.concatenate(
        [dst.reshape(NW, epw), trash], axis=1).reshape(NW, k_steps, CHUNK)
  else:
    trash = n + (jnp.arange(pad, dtype=jnp.int32) % 128)
    srcp = jnp.concatenate([src, jnp.zeros((pad,), jnp.int32)]).reshape(
        NW, k_steps, CHUNK)
    dstp = jnp.concatenate([dst, trash]).reshape(NW, k_steps, CHUNK)
  agg = _make_sc_agg(n_pad, d, k_steps)
  cntp = _make_sc_cnt(n_pad, k_steps)(dstp)

  part1 = agg(node_features, srcp, dstp)
  b1r = b1.reshape(1, d)
  b2r = b2.reshape(1, d)
  h1 = _tc_dense(part1, cntp, node_features, W1_l, W1_r, b1r,
                 relu=True, n=n, d=d)
  part2 = agg(h1, srcp, dstp)
  out = _tc_dense(part2, cntp, h1, W2_l, W2_r, b2r, relu=False, n=n, d=d)
  return out
